# baseline (speedup 1.0000x reference)
import functools

import jax
import jax.numpy as jnp
from jax import lax
from jax.experimental import pallas as pl
from jax.experimental.pallas import tpu as pltpu
from jax.experimental.pallas import tpu_sc as plsc

NC = 2
NS = 16
NW = NC * NS

B = 4096 * 200
D = 64
DW = 128

CHUNK = 128
NBUF = 5
B_PER_W = B // NW
SUPER = NBUF * CHUNK
N_SUPER = B_PER_W // SUPER
assert B_PER_W % SUPER == 0


VOCAB = 1000000
VB = 1024


def _widen_table(table):
    tT = table.T

    def prep(tT_ref, out_ref):
        blk = tT_ref[...]
        out_ref[:, :D] = blk.T
        out_ref[:, D:] = jnp.zeros((VB, DW - D), jnp.float32)

    return pl.pallas_call(
        prep,
        grid=(pl.cdiv(VOCAB, VB),),
        in_specs=[pl.BlockSpec((D, VB), lambda i: (0, i))],
        out_specs=pl.BlockSpec((VB, DW), lambda i: (i, 0)),
        out_shape=jax.ShapeDtypeStruct((VOCAB, DW), jnp.float32),
    )(tT)


def kernel(x, table):
    idx = x.reshape(B // CHUNK, CHUNK).astype(jnp.int32)
    tablew = _widen_table(table)
    mesh = plsc.VectorSubcoreMesh(core_axis_name="c", subcore_axis_name="s")

    @functools.partial(
        pl.kernel,
        mesh=mesh,
        out_type=jax.ShapeDtypeStruct((B, DW), jnp.float32),
        scratch_types=[
            pltpu.VMEM((NBUF, CHUNK), jnp.int32),
            pltpu.VMEM((NBUF, CHUNK, DW), jnp.float32),
        ]
        + [pltpu.SemaphoreType.DMA] * (2 * NBUF),
        compiler_params=pltpu.CompilerParams(use_tc_tiling_on_sc=False),
    )
    def gather_kernel(idx_hbm, table_hbm, out_hbm, idx_v, rows_v, *sems):
        gsems = sems[:NBUF]
        osems = sems[NBUF:]
        wid = lax.axis_index("s") * NC + lax.axis_index("c")
        idx_row_base = wid * (N_SUPER * NBUF)
        out_base = wid * B_PER_W

        @pl.loop(0, N_SUPER)
        def _(t):
            pltpu.sync_copy(
                idx_hbm.at[pl.ds(idx_row_base + t * NBUF, NBUF)],
                idx_v,
            )
            handles = []
            for b in range(NBUF):
                @pl.when(t > 0)
                def _(b=b):
                    pltpu.make_async_copy(
                        rows_v.at[b],
                        out_hbm.at[pl.ds(out_base, CHUNK)],
                        osems[b],
                    ).wait()

                handles.append(
                    pltpu.async_copy(
                        table_hbm.at[idx_v.at[b]],
                        rows_v.at[b],
                        gsems[b],
                    )
                )
            for b in range(NBUF):
                handles[b].wait()
                pltpu.async_copy(
                    rows_v.at[b],
                    out_hbm.at[pl.ds(out_base + (t * NBUF + b) * CHUNK, CHUNK)],
                    osems[b],
                )

        for b in range(NBUF):
            pltpu.make_async_copy(
                rows_v.at[b],
                out_hbm.at[pl.ds(out_base, CHUNK)],
                osems[b],
            ).wait()

    outw = gather_kernel(idx, tablew)
    return outw[:, :D].reshape(x.shape[0], x.shape[1], D)


# speedup vs baseline: 1.3624x; 1.3624x over previous
import functools

import jax
import jax.numpy as jnp
from jax import lax
from jax.experimental import pallas as pl
from jax.experimental.pallas import tpu as pltpu
from jax.experimental.pallas import tpu_sc as plsc

NC = 2
NS = 16
NW = NC * NS

B = 4096 * 200
D = 64
DW = 128
VOCAB = 1000000

CHUNK = 128
NBUF = 5
B_PER_W = B // NW
SUPER = NBUF * CHUNK
N_SUPER = B_PER_W // SUPER
assert B_PER_W % SUPER == 0


def kernel(x, table):
    idx = (x * 2).reshape(B // CHUNK, CHUNK).astype(jnp.int32)
    tablew = jnp.concatenate(
        [table, jnp.zeros((VOCAB, DW - D), jnp.float32)], axis=1
    )
    table2 = tablew.reshape(2 * VOCAB, D)
    mesh = plsc.VectorSubcoreMesh(core_axis_name="c", subcore_axis_name="s")

    @functools.partial(
        pl.kernel,
        mesh=mesh,
        out_type=jax.ShapeDtypeStruct((B, DW), jnp.float32),
        scratch_types=[
            pltpu.VMEM((NBUF, CHUNK), jnp.int32),
            pltpu.VMEM((NBUF, CHUNK, D), jnp.float32),
        ]
        + [pltpu.SemaphoreType.DMA] * (2 * NBUF),
        compiler_params=pltpu.CompilerParams(use_tc_tiling_on_sc=False),
    )
    def gather_kernel(idx_hbm, table_hbm, out_hbm, idx_v, rows_v, *sems):
        gsems = sems[:NBUF]
        osems = sems[NBUF:]
        wid = lax.axis_index("s") * NC + lax.axis_index("c")
        idx_row_base = wid * (N_SUPER * NBUF)
        out_base = wid * B_PER_W

        @pl.loop(0, N_SUPER)
        def _(t):
            pltpu.sync_copy(
                idx_hbm.at[pl.ds(idx_row_base + t * NBUF, NBUF)],
                idx_v,
            )
            handles = []
            for b in range(NBUF):
                @pl.when(t > 0)
                def _(b=b):
                    pltpu.make_async_copy(
                        rows_v.at[b],
                        out_hbm.at[pl.ds(out_base, CHUNK), pl.ds(0, D)],
                        osems[b],
                    ).wait()

                handles.append(
                    pltpu.async_copy(
                        table_hbm.at[idx_v.at[b]],
                        rows_v.at[b],
                        gsems[b],
                    )
                )
            for b in range(NBUF):
                handles[b].wait()
                pltpu.async_copy(
                    rows_v.at[b],
                    out_hbm.at[
                        pl.ds(out_base + (t * NBUF + b) * CHUNK, CHUNK),
                        pl.ds(0, D),
                    ],
                    osems[b],
                )

        for b in range(NBUF):
            pltpu.make_async_copy(
                rows_v.at[b],
                out_hbm.at[pl.ds(out_base, CHUNK), pl.ds(0, D)],
                osems[b],
            ).wait()

    outw = gather_kernel(idx, table2)
    return outw[:, :D].reshape(x.shape[0], x.shape[1], D)


# speedup vs baseline: 1.3810x; 1.0136x over previous
import functools

import jax
import jax.numpy as jnp
from jax import lax
from jax.experimental import pallas as pl
from jax.experimental.pallas import tpu as pltpu
from jax.experimental.pallas import tpu_sc as plsc

NC = 2
NS = 16
NW = NC * NS

B = 4096 * 200
D = 64
DW = 128
VOCAB = 1000000

CHUNK = 128
NBUF = 8
B_PER_W = B // NW
SUPER = NBUF * CHUNK
N_SUPER = B_PER_W // SUPER
assert B_PER_W % SUPER == 0


def kernel(x, table):
    idx = (x * 2).reshape(B // CHUNK, CHUNK).astype(jnp.int32)
    tablew = jnp.concatenate(
        [table, jnp.zeros((VOCAB, DW - D), jnp.float32)], axis=1
    )
    table2 = tablew.reshape(2 * VOCAB, D)
    mesh = plsc.VectorSubcoreMesh(core_axis_name="c", subcore_axis_name="s")

    @functools.partial(
        pl.kernel,
        mesh=mesh,
        out_type=jax.ShapeDtypeStruct((B, DW), jnp.float32),
        scratch_types=[
            pltpu.VMEM((NBUF, CHUNK), jnp.int32),
            pltpu.VMEM((NBUF, CHUNK, D), jnp.float32),
        ]
        + [pltpu.SemaphoreType.DMA] * (2 * NBUF),
        compiler_params=pltpu.CompilerParams(use_tc_tiling_on_sc=False),
    )
    def gather_kernel(idx_hbm, table_hbm, out_hbm, idx_v, rows_v, *sems):
        gsems = sems[:NBUF]
        osems = sems[NBUF:]
        wid = lax.axis_index("s") * NC + lax.axis_index("c")
        idx_row_base = wid * (N_SUPER * NBUF)
        out_base = wid * B_PER_W

        @pl.loop(0, N_SUPER)
        def _(t):
            pltpu.sync_copy(
                idx_hbm.at[pl.ds(idx_row_base + t * NBUF, NBUF)],
                idx_v,
            )
            handles = []
            for b in range(NBUF):
                @pl.when(t > 0)
                def _(b=b):
                    pltpu.make_async_copy(
                        rows_v.at[b],
                        out_hbm.at[pl.ds(out_base, CHUNK), pl.ds(0, D)],
                        osems[b],
                    ).wait()

                handles.append(
                    pltpu.async_copy(
                        table_hbm.at[idx_v.at[b]],
                        rows_v.at[b],
                        gsems[b],
                    )
                )
            for b in range(NBUF):
                handles[b].wait()
                pltpu.async_copy(
                    rows_v.at[b],
                    out_hbm.at[
                        pl.ds(out_base + (t * NBUF + b) * CHUNK, CHUNK),
                        pl.ds(0, D),
                    ],
                    osems[b],
                )

        for b in range(NBUF):
            pltpu.make_async_copy(
                rows_v.at[b],
                out_hbm.at[pl.ds(out_base, CHUNK), pl.ds(0, D)],
                osems[b],
            ).wait()

    outw = gather_kernel(idx, table2)
    return outw[:, :D].reshape(x.shape[0], x.shape[1], D)


# speedup vs baseline: 1.4020x; 1.0153x over previous
import functools

import jax
import jax.numpy as jnp
from jax import lax
from jax.experimental import pallas as pl
from jax.experimental.pallas import tpu as pltpu
from jax.experimental.pallas import tpu_sc as plsc

NC = 2
NS = 16
NW = NC * NS

B = 4096 * 200
D = 64
DW = 128
VOCAB = 1000000

CHUNK = 128
NBUF = 8
B_PER_W = B // NW
SUPER = NBUF * CHUNK
N_SUPER = B_PER_W // SUPER
assert B_PER_W % SUPER == 0


def kernel(x, table):
    idx = (x * 2).reshape(B // CHUNK, CHUNK).astype(jnp.int32)
    tablew = jnp.concatenate(
        [table, jnp.zeros((VOCAB, DW - D), jnp.float32)], axis=1
    )
    table2 = tablew.reshape(2 * VOCAB, D)
    mesh = plsc.VectorSubcoreMesh(core_axis_name="c", subcore_axis_name="s")

    @functools.partial(
        pl.kernel,
        mesh=mesh,
        out_type=jax.ShapeDtypeStruct((B, DW), jnp.float32),
        scratch_types=[
            pltpu.VMEM((2, NBUF, CHUNK), jnp.int32),
            pltpu.VMEM((NBUF, CHUNK, D), jnp.float32),
        ]
        + [pltpu.SemaphoreType.DMA] * (2 * NBUF + 2),
        compiler_params=pltpu.CompilerParams(use_tc_tiling_on_sc=False),
    )
    def gather_kernel(idx_hbm, table_hbm, out_hbm, idx_v, rows_v, *sems):
        gsems = sems[:NBUF]
        osems = sems[NBUF : 2 * NBUF]
        isems = sems[2 * NBUF :]
        wid = lax.axis_index("s") * NC + lax.axis_index("c")
        idx_row_base = wid * (N_SUPER * NBUF)
        out_base = wid * B_PER_W

        pltpu.async_copy(
            idx_hbm.at[pl.ds(idx_row_base, NBUF)], idx_v.at[0], isems[0]
        )

        @pl.loop(0, N_SUPER)
        def _(t):
            p = lax.rem(t, 2)
            pltpu.make_async_copy(
                idx_hbm.at[pl.ds(idx_row_base, NBUF)], idx_v.at[p], isems[0]
            ).wait()
            @pl.when(t + 1 < N_SUPER)
            def _():
                pltpu.async_copy(
                    idx_hbm.at[pl.ds(idx_row_base + (t + 1) * NBUF, NBUF)],
                    idx_v.at[1 - p],
                    isems[0],
                )

            handles = []
            for b in range(NBUF):
                @pl.when(t > 0)
                def _(b=b):
                    pltpu.make_async_copy(
                        rows_v.at[b],
                        out_hbm.at[pl.ds(out_base, CHUNK), pl.ds(0, D)],
                        osems[b],
                    ).wait()

                handles.append(
                    pltpu.async_copy(
                        table_hbm.at[idx_v.at[p].at[b]],
                        rows_v.at[b],
                        gsems[b],
                    )
                )
            for b in range(NBUF):
                handles[b].wait()
                pltpu.async_copy(
                    rows_v.at[b],
                    out_hbm.at[
                        pl.ds(out_base + (t * NBUF + b) * CHUNK, CHUNK),
                        pl.ds(0, D),
                    ],
                    osems[b],
                )

        for b in range(NBUF):
            pltpu.make_async_copy(
                rows_v.at[b],
                out_hbm.at[pl.ds(out_base, CHUNK), pl.ds(0, D)],
                osems[b],
            ).wait()

    outw = gather_kernel(idx, table2)
    return outw[:, :D].reshape(x.shape[0], x.shape[1], D)
